# baseline (speedup 1.0000x reference)
import dataclasses
import functools

import jax
import jax.numpy as jnp
from jax import lax
from jax.experimental import pallas as pl
from jax.experimental.pallas import tpu as pltpu
from jax.experimental.pallas import tpu_sc as plsc

_B = 16384
_D = 64
_V = 100000
_NC = 2
_NS = 16
_L = 16
_NW = _NC * _NS
_FPW = _D // _NW
_BCHUNK = 4096
_NBCH = _B // _BCHUNK


def _sc_compiler_params():
    cp = pltpu.CompilerParams()
    if "needs_layout_passes" in pltpu.CompilerParams.__dataclass_fields__:
        cp = dataclasses.replace(cp, needs_layout_passes=False)
    return cp


def _make_gather_kernel():
    mesh = plsc.VectorSubcoreMesh(
        core_axis_name="c", subcore_axis_name="s",
        num_cores=_NC, num_subcores=_NS,
    )

    @functools.partial(
        pl.kernel,
        out_type=jax.ShapeDtypeStruct((_D, _B), jnp.float32),
        mesh=mesh,
        scratch_types=[
            pltpu.VMEM((_V,), jnp.float32),
            pltpu.VMEM((_B,), jnp.int32),
            pltpu.VMEM((_BCHUNK,), jnp.float32),
            pltpu.VMEM((_BCHUNK,), jnp.float32),
            pltpu.SemaphoreType.DMA,
            pltpu.SemaphoreType.DMA,
            pltpu.SemaphoreType.DMA,
        ],
        compiler_params=_sc_compiler_params(),
    )
    def gather_kernel(ct_hbm, lab_hbm, out_hbm, row_v, lab_v, o0_v, o1_v,
                      rsem, lsem, osem):
        o_bufs = (o0_v, o1_v)
        wid = lax.axis_index("s") * _NC + lax.axis_index("c")
        lcp = pltpu.async_copy(lab_hbm, lab_v, lsem)
        rcp = pltpu.async_copy(ct_hbm.at[wid * _FPW], row_v, rsem)
        lcp.wait()
        ocps = [None, None]
        for t in range(_FPW):
            f = wid * _FPW + t
            rcp.wait()
            for k in range(_NBCH):
                buf = k % 2
                if ocps[buf] is not None:
                    ocps[buf].wait()

                @plsc.parallel_loop(0, _BCHUNK, step=8 * _L, unroll=2)
                def _(j):
                    idxs = [lab_v[pl.ds(k * _BCHUNK + j + u * _L, _L)]
                            for u in range(8)]
                    gs = [plsc.load_gather(row_v, [idxs[u]]) for u in range(8)]
                    for u in range(8):
                        o_bufs[buf][pl.ds(j + u * _L, _L)] = gs[u]

                if t + 1 == _FPW and k + 1 == _NBCH:
                    pass
                elif k + 1 == _NBCH:
                    rcp = pltpu.async_copy(ct_hbm.at[f + 1], row_v, rsem)
                ocps[buf] = pltpu.async_copy(
                    o_bufs[buf],
                    out_hbm.at[f, pl.ds(k * _BCHUNK, _BCHUNK)],
                    osem)
        for cp in ocps:
            if cp is not None:
                cp.wait()

    return gather_kernel


_GATHER = _make_gather_kernel()

_BC = 2048


def _loss_body(xt_ref, g_ref, o_ref):
    i = pl.program_id(0)
    d = xt_ref[...] - g_ref[...]
    s = jnp.sum(d * d, axis=0, keepdims=True)
    s = jnp.minimum(jnp.maximum(s, 1e-12), 1e12)
    part = jnp.sum(s) * (1.0 / _B)

    @pl.when(i == 0)
    def _():
        o_ref[...] = jnp.zeros((1, 128), jnp.float32)

    o_ref[...] += jnp.full((1, 128), part, jnp.float32)


_LOSS = pl.pallas_call(
    _loss_body,
    out_shape=jax.ShapeDtypeStruct((1, 128), jnp.float32),
    grid=(_B // _BC,),
    in_specs=[
        pl.BlockSpec((_D, _BC), lambda i: (0, i)),
        pl.BlockSpec((_D, _BC), lambda i: (0, i)),
    ],
    out_specs=pl.BlockSpec((1, 128), lambda i: (0, 0)),
)


def kernel(x, labels, centers):
    xt = x.T
    ct = centers.T
    lab = labels.astype(jnp.int32)
    g = _GATHER(ct, lab)
    return _LOSS(xt, g)[0, 0]


# speedup vs baseline: 1.0571x; 1.0571x over previous
import dataclasses
import functools

import jax
import jax.numpy as jnp
from jax import lax
from jax.experimental import pallas as pl
from jax.experimental.pallas import tpu as pltpu
from jax.experimental.pallas import tpu_sc as plsc

_B = 16384
_D = 64
_V = 100000
_NC = 2
_NS = 16
_L = 16
_NW = _NC * _NS
_FPW = _D // _NW
_BCHUNK = 4096
_NBCH = _B // _BCHUNK


def _sc_compiler_params():
    cp = pltpu.CompilerParams()
    if "needs_layout_passes" in pltpu.CompilerParams.__dataclass_fields__:
        cp = dataclasses.replace(cp, needs_layout_passes=False)
    return cp


def _make_gather_kernel():
    mesh = plsc.VectorSubcoreMesh(
        core_axis_name="c", subcore_axis_name="s",
        num_cores=_NC, num_subcores=_NS,
    )

    @functools.partial(
        pl.kernel,
        out_type=jax.ShapeDtypeStruct((_D, _B), jnp.float32),
        mesh=mesh,
        scratch_types=[
            pltpu.VMEM((_V,), jnp.float32),
            pltpu.VMEM((_B,), jnp.int32),
            pltpu.VMEM((_BCHUNK,), jnp.float32),
            pltpu.VMEM((_BCHUNK,), jnp.float32),
            pltpu.SemaphoreType.DMA,
            pltpu.SemaphoreType.DMA,
            pltpu.SemaphoreType.DMA,
        ],
        compiler_params=_sc_compiler_params(),
    )
    def gather_kernel(ct_hbm, lab_hbm, out_hbm, row_v, lab_v, o0_v, o1_v,
                      rsem, lsem, osem):
        o_bufs = (o0_v, o1_v)
        wid = lax.axis_index("s") * _NC + lax.axis_index("c")
        lcp = pltpu.async_copy(lab_hbm, lab_v, lsem)
        rcp = pltpu.async_copy(ct_hbm.at[wid * _FPW], row_v, rsem)
        lcp.wait()
        ocps = [None, None]
        for t in range(_FPW):
            f = wid * _FPW + t
            rcp.wait()
            for k in range(_NBCH):
                buf = k % 2
                if ocps[buf] is not None:
                    ocps[buf].wait()

                @pl.loop(0, _BCHUNK, step=16 * _L)
                def _(j):
                    idxs = [lab_v[pl.ds(k * _BCHUNK + j + u * _L, _L)]
                            for u in range(16)]
                    gs = [plsc.load_gather(row_v, [idxs[u]])
                          for u in range(16)]
                    for u in range(16):
                        o_bufs[buf][pl.ds(j + u * _L, _L)] = gs[u]

                if t + 1 == _FPW and k + 1 == _NBCH:
                    pass
                elif k + 1 == _NBCH:
                    rcp = pltpu.async_copy(ct_hbm.at[f + 1], row_v, rsem)
                ocps[buf] = pltpu.async_copy(
                    o_bufs[buf],
                    out_hbm.at[f, pl.ds(k * _BCHUNK, _BCHUNK)],
                    osem)
        for cp in ocps:
            if cp is not None:
                cp.wait()

    return gather_kernel


_GATHER = _make_gather_kernel()

_BC = 4096


def _loss_body(xt_ref, g_ref, o_ref):
    i = pl.program_id(0)
    d = xt_ref[...] - g_ref[...]
    s = jnp.sum(d * d, axis=0, keepdims=True)
    s = jnp.minimum(jnp.maximum(s, 1e-12), 1e12)
    part = jnp.sum(s) * (1.0 / _B)

    @pl.when(i == 0)
    def _():
        o_ref[...] = jnp.zeros((1, 128), jnp.float32)

    o_ref[...] += jnp.full((1, 128), part, jnp.float32)


_LOSS = pl.pallas_call(
    _loss_body,
    out_shape=jax.ShapeDtypeStruct((1, 128), jnp.float32),
    grid=(_B // _BC,),
    in_specs=[
        pl.BlockSpec((_D, _BC), lambda i: (0, i)),
        pl.BlockSpec((_D, _BC), lambda i: (0, i)),
    ],
    out_specs=pl.BlockSpec((1, 128), lambda i: (0, 0)),
)


def kernel(x, labels, centers):
    xt = x.T
    ct = centers.T
    lab = labels.astype(jnp.int32)
    g = _GATHER(ct, lab)
    return _LOSS(xt, g)[0, 0]


# speedup vs baseline: 1.0724x; 1.0145x over previous
import dataclasses
import functools

import jax
import jax.numpy as jnp
from jax import lax
from jax.experimental import pallas as pl
from jax.experimental.pallas import tpu as pltpu
from jax.experimental.pallas import tpu_sc as plsc

_B = 16384
_D = 64
_V = 100000
_NC = 2
_NS = 16
_L = 16
_NW = _NC * _NS
_FPW = _D // _NW
_BCHUNK = 4096
_NBCH = _B // _BCHUNK


def _sc_compiler_params():
    cp = pltpu.CompilerParams()
    if "needs_layout_passes" in pltpu.CompilerParams.__dataclass_fields__:
        cp = dataclasses.replace(cp, needs_layout_passes=False)
    return cp


def _make_gather_kernel():
    mesh = plsc.VectorSubcoreMesh(
        core_axis_name="c", subcore_axis_name="s",
        num_cores=_NC, num_subcores=_NS,
    )

    @functools.partial(
        pl.kernel,
        out_type=jax.ShapeDtypeStruct((_D, _B), jnp.float32),
        mesh=mesh,
        scratch_types=[
            pltpu.VMEM((_V,), jnp.float32),
            pltpu.VMEM((_B,), jnp.int32),
            pltpu.VMEM((_BCHUNK,), jnp.float32),
            pltpu.VMEM((_BCHUNK,), jnp.float32),
            pltpu.SemaphoreType.DMA,
            pltpu.SemaphoreType.DMA,
            pltpu.SemaphoreType.DMA,
        ],
        compiler_params=_sc_compiler_params(),
    )
    def gather_kernel(ct_hbm, lab_hbm, out_hbm, row_v, lab_v, o0_v, o1_v,
                      rsem, lsem, osem):
        o_bufs = (o0_v, o1_v)
        wid = lax.axis_index("s") * _NC + lax.axis_index("c")
        rcp = pltpu.async_copy(ct_hbm.at[wid * _FPW], row_v, rsem)
        lcp = pltpu.async_copy(lab_hbm, lab_v, lsem)
        ocps = [None, None]
        for t in range(_FPW):
            f = wid * _FPW + t
            rcp.wait()
            if t == 0:
                lcp.wait()
            for k in range(_NBCH):
                buf = k % 2
                if ocps[buf] is not None:
                    ocps[buf].wait()

                @pl.loop(0, _BCHUNK, step=16 * _L)
                def _(j):
                    idxs = [lab_v[pl.ds(k * _BCHUNK + j + u * _L, _L)]
                            for u in range(16)]
                    gs = [plsc.load_gather(row_v, [idxs[u]])
                          for u in range(16)]
                    for u in range(16):
                        o_bufs[buf][pl.ds(j + u * _L, _L)] = gs[u]

                if t + 1 == _FPW and k + 1 == _NBCH:
                    pass
                elif k + 1 == _NBCH:
                    rcp = pltpu.async_copy(ct_hbm.at[f + 1], row_v, rsem)
                ocps[buf] = pltpu.async_copy(
                    o_bufs[buf],
                    out_hbm.at[f, pl.ds(k * _BCHUNK, _BCHUNK)],
                    osem)
        for cp in ocps:
            if cp is not None:
                cp.wait()

    return gather_kernel


_GATHER = _make_gather_kernel()

_BC = 8192


def _loss_body(xt_ref, g_ref, o_ref):
    i = pl.program_id(0)
    d = xt_ref[...] - g_ref[...]
    s = jnp.sum(d * d, axis=0, keepdims=True)
    s = jnp.minimum(jnp.maximum(s, 1e-12), 1e12)
    part = jnp.sum(s) * (1.0 / _B)

    @pl.when(i == 0)
    def _():
        o_ref[...] = jnp.zeros((1, 128), jnp.float32)

    o_ref[...] += jnp.full((1, 128), part, jnp.float32)


_LOSS = pl.pallas_call(
    _loss_body,
    out_shape=jax.ShapeDtypeStruct((1, 128), jnp.float32),
    grid=(_B // _BC,),
    in_specs=[
        pl.BlockSpec((_D, _BC), lambda i: (0, i)),
        pl.BlockSpec((_D, _BC), lambda i: (0, i)),
    ],
    out_specs=pl.BlockSpec((1, 128), lambda i: (0, 0)),
)


def kernel(x, labels, centers):
    xt = x.T
    ct = centers.T
    lab = labels.astype(jnp.int32)
    g = _GATHER(ct, lab)
    return _LOSS(xt, g)[0, 0]


# speedup vs baseline: 1.0755x; 1.0029x over previous
import dataclasses
import functools

import jax
import jax.numpy as jnp
from jax import lax
from jax.experimental import pallas as pl
from jax.experimental.pallas import tpu as pltpu
from jax.experimental.pallas import tpu_sc as plsc

_B = 16384
_D = 64
_V = 100000
_NC = 2
_NS = 16
_L = 16
_NW = _NC * _NS
_FPW = _D // _NW
_BCHUNK = 4096
_NBCH = _B // _BCHUNK


def _sc_compiler_params():
    cp = pltpu.CompilerParams(
        disable_bounds_checks=True,
        disable_semaphore_checks=True,
    )
    if "needs_layout_passes" in pltpu.CompilerParams.__dataclass_fields__:
        cp = dataclasses.replace(cp, needs_layout_passes=False)
    return cp


def _make_gather_kernel():
    mesh = plsc.VectorSubcoreMesh(
        core_axis_name="c", subcore_axis_name="s",
        num_cores=_NC, num_subcores=_NS,
    )

    @functools.partial(
        pl.kernel,
        out_type=jax.ShapeDtypeStruct((_D, _B), jnp.float32),
        mesh=mesh,
        scratch_types=[
            pltpu.VMEM((_V,), jnp.float32),
            pltpu.VMEM((_B,), jnp.int32),
            pltpu.VMEM((_BCHUNK,), jnp.float32),
            pltpu.VMEM((_BCHUNK,), jnp.float32),
            pltpu.SemaphoreType.DMA,
            pltpu.SemaphoreType.DMA,
            pltpu.SemaphoreType.DMA,
        ],
        compiler_params=_sc_compiler_params(),
    )
    def gather_kernel(ct_hbm, lab_hbm, out_hbm, row_v, lab_v, o0_v, o1_v,
                      rsem, lsem, osem):
        o_bufs = (o0_v, o1_v)
        wid = lax.axis_index("s") * _NC + lax.axis_index("c")
        rcp = pltpu.async_copy(ct_hbm.at[wid * _FPW], row_v, rsem)
        lcp = pltpu.async_copy(lab_hbm, lab_v, lsem)
        ocps = [None, None]
        for t in range(_FPW):
            f = wid * _FPW + t
            rcp.wait()
            if t == 0:
                lcp.wait()
            for k in range(_NBCH):
                buf = k % 2
                if ocps[buf] is not None:
                    ocps[buf].wait()

                @pl.loop(0, _BCHUNK, step=16 * _L)
                def _(j):
                    idxs = [lab_v[pl.ds(k * _BCHUNK + j + u * _L, _L)]
                            for u in range(16)]
                    gs = [plsc.load_gather(row_v, [idxs[u]])
                          for u in range(16)]
                    for u in range(16):
                        o_bufs[buf][pl.ds(j + u * _L, _L)] = gs[u]

                if t + 1 == _FPW and k + 1 == _NBCH:
                    pass
                elif k + 1 == _NBCH:
                    rcp = pltpu.async_copy(ct_hbm.at[f + 1], row_v, rsem)
                ocps[buf] = pltpu.async_copy(
                    o_bufs[buf],
                    out_hbm.at[f, pl.ds(k * _BCHUNK, _BCHUNK)],
                    osem)
        for cp in ocps:
            if cp is not None:
                cp.wait()

    return gather_kernel


_GATHER = _make_gather_kernel()

_BC = 8192


def _loss_body(xt_ref, g_ref, o_ref):
    i = pl.program_id(0)
    d = xt_ref[...] - g_ref[...]
    s = jnp.sum(d * d, axis=0, keepdims=True)
    s = jnp.minimum(jnp.maximum(s, 1e-12), 1e12)
    part = jnp.sum(s) * (1.0 / _B)

    @pl.when(i == 0)
    def _():
        o_ref[...] = jnp.zeros((1, 128), jnp.float32)

    o_ref[...] += jnp.full((1, 128), part, jnp.float32)


_LOSS = pl.pallas_call(
    _loss_body,
    out_shape=jax.ShapeDtypeStruct((1, 128), jnp.float32),
    grid=(_B // _BC,),
    in_specs=[
        pl.BlockSpec((_D, _BC), lambda i: (0, i)),
        pl.BlockSpec((_D, _BC), lambda i: (0, i)),
    ],
    out_specs=pl.BlockSpec((1, 128), lambda i: (0, 0)),
)


def kernel(x, labels, centers):
    xt = x.T
    ct = centers.T
    lab = labels.astype(jnp.int32)
    g = _GATHER(ct, lab)
    return _LOSS(xt, g)[0, 0]
